# baseline (speedup 1.0000x reference)
import functools

import jax
import jax.numpy as jnp
from jax import lax
from jax.experimental import pallas as pl
from jax.experimental.pallas import tpu as pltpu
from jax.experimental.pallas import tpu_sc as plsc

_NC = 2
_NS = 16
_NW = _NC * _NS
_LANES = 128
_F0 = 0.60


def _pre_body(x_ref, wm_ref, bm_ref, wu2_ref, bu_ref, y_ref, z_ref):
    xb = x_ref[...]
    y_ref[...] = jnp.maximum(
        jnp.dot(xb, wm_ref[...], preferred_element_type=jnp.float32) + bm_ref[...],
        0.0)
    z_ref[...] = jnp.dot(xb, wu2_ref[...], preferred_element_type=jnp.float32) + bu_ref[...]


def _pre(x, Wm, bm2, Wu2, bu2):
    n, d = x.shape
    blk = 2000
    grid = n // blk
    return pl.pallas_call(
        _pre_body,
        grid=(grid,),
        in_specs=[
            pl.BlockSpec((blk, d), lambda i: (i, 0)),
            pl.BlockSpec((d, d), lambda i: (0, 0)),
            pl.BlockSpec((1, d), lambda i: (0, 0)),
            pl.BlockSpec((d, d), lambda i: (0, 0)),
            pl.BlockSpec((1, d), lambda i: (0, 0)),
        ],
        out_specs=[
            pl.BlockSpec((blk, d), lambda i: (i, 0)),
            pl.BlockSpec((blk, d), lambda i: (i, 0)),
        ],
        out_shape=[
            jax.ShapeDtypeStruct((n, d), jnp.float32),
            jax.ShapeDtypeStruct((n, d), jnp.float32),
        ],
    )(x, Wm, bm2, Wu2, bu2)


def _post_body(p0_ref, p1_ref, z_ref, wu1_ref, h_ref):
    agg = p0_ref[...] + p1_ref[...]
    h_ref[...] = jnp.maximum(
        jnp.dot(agg, wu1_ref[...], preferred_element_type=jnp.float32) + z_ref[...],
        0.0)


def _post(p0, p1, z, Wu1):
    n, d = z.shape
    blk = 2000
    grid = n // blk
    return pl.pallas_call(
        _post_body,
        grid=(grid,),
        in_specs=[
            pl.BlockSpec((blk, d), lambda i: (i, 0)),
            pl.BlockSpec((blk, d), lambda i: (i, 0)),
            pl.BlockSpec((blk, d), lambda i: (i, 0)),
            pl.BlockSpec((d, d), lambda i: (0, 0)),
        ],
        out_specs=pl.BlockSpec((blk, d), lambda i: (i, 0)),
        out_shape=jax.ShapeDtypeStruct((n, d), jnp.float32),
    )(p0, p1, z, Wu1)


def _sc_scatter(y, ei_r, zeros_pad, n, d, n_pad, k0, c0_chunks, base1, rem1,
                kmax):
    rows_out = n_pad // _NS

    def body(y_hbm, ei_hbm, zero_hbm, out_hbm, idx_s, idx_d, rows,
             agg_sh, sem):
        c = lax.axis_index("c")
        s = lax.axis_index("s")

        nch = jnp.where(c == 0, k0, base1 + jnp.where(s < rem1, 1, 0))
        start = pl.multiple_of(
            jnp.where(c == 0, s * k0,
                      c0_chunks + base1 * s + jnp.minimum(s, rem1)), 8)

        pltpu.sync_copy(zero_hbm.at[pl.ds(s * rows_out, rows_out)],
                        agg_sh.at[pl.ds(s * rows_out, rows_out)])
        pltpu.sync_copy(ei_hbm.at[0, pl.ds(start, kmax)], idx_s)
        pltpu.sync_copy(ei_hbm.at[1, pl.ds(start, kmax)], idx_d)
        plsc.subcore_barrier()

        def step(j, carry):
            pltpu.async_copy(y_hbm.at[idx_s.at[j]], rows, sem).wait()
            pltpu.sync_copy(rows, agg_sh.at[idx_d.at[j]], add=True)
            return carry

        lax.fori_loop(0, nch, step, 0, unroll=False)
        plsc.subcore_barrier()

        pltpu.sync_copy(agg_sh.at[pl.ds(s * rows_out, rows_out)],
                        out_hbm.at[c, pl.ds(s * rows_out, rows_out)])

    mesh = plsc.VectorSubcoreMesh(core_axis_name="c", subcore_axis_name="s")
    f = pl.kernel(
        body,
        out_type=jax.ShapeDtypeStruct((_NC, n_pad, d), jnp.float32),
        mesh=mesh,
        scratch_types=[
            pltpu.VMEM((kmax, _LANES), jnp.int32),
            pltpu.VMEM((kmax, _LANES), jnp.int32),
            pltpu.VMEM((_LANES, d), jnp.float32),
            pltpu.VMEM_SHARED((n_pad, d), jnp.float32),
            pltpu.SemaphoreType.DMA,
        ],
    )
    return f(y, ei_r, zeros_pad)


def kernel(x, edge_index, Wm, bm, Wu, bu):
    n, d = x.shape
    e = edge_index.shape[1]

    p = -(-e // (_LANES * 128)) * 128
    c0_chunks = max(_NS * 8, int(round(_F0 * p / 128)) * 128)
    c1_chunks = p - c0_chunks
    k0 = c0_chunks // _NS
    base1 = c1_chunks // _NS
    rem1 = 0
    kmax = max(k0, base1)
    p_pad = c0_chunks + base1 * (_NS - 1) + kmax
    n_pad = -(-(n + 1) // (_NS * 8)) * (_NS * 8)

    pad = p_pad * _LANES - e
    pad_cols = jnp.concatenate(
        [jnp.zeros((1, pad), jnp.int32), jnp.full((1, pad), n, jnp.int32)])
    ei_r = jnp.concatenate([edge_index, pad_cols], axis=1).reshape(
        2, p_pad, _LANES)
    zeros_pad = jnp.zeros((n_pad, d), jnp.float32)

    bm2 = bm.reshape(1, d)
    bu2 = bu.reshape(1, d)
    Wu1 = Wu[:d]
    Wu2 = Wu[d:]

    y, z = _pre(x, Wm, bm2, Wu2, bu2)
    parts = _sc_scatter(y, ei_r, zeros_pad, n, d, n_pad, k0, c0_chunks,
                        base1, rem1, kmax)
    h = _post(parts[0, :n], parts[1, :n], z, Wu1)
    return h


# speedup vs baseline: 2.2434x; 2.2434x over previous
import functools

import jax
import jax.numpy as jnp
from jax import lax
from jax.experimental import pallas as pl
from jax.experimental.pallas import tpu as pltpu
from jax.experimental.pallas import tpu_sc as plsc

_NC = 2
_NS = 16
_NW = _NC * _NS
_LANES = 128
_F0 = 0.60


def _pre_body(x_ref, wm_ref, bm_ref, wu2_ref, bu_ref, y_ref, z_ref):
    xb = x_ref[...]
    y_ref[...] = jnp.maximum(
        jnp.dot(xb, wm_ref[...], preferred_element_type=jnp.float32) + bm_ref[...],
        0.0)
    z_ref[...] = jnp.dot(xb, wu2_ref[...], preferred_element_type=jnp.float32) + bu_ref[...]


def _pre(x, Wm, bm2, Wu2, bu2):
    n, d = x.shape
    blk = 2000
    grid = n // blk
    return pl.pallas_call(
        _pre_body,
        grid=(grid,),
        in_specs=[
            pl.BlockSpec((blk, d), lambda i: (i, 0)),
            pl.BlockSpec((d, d), lambda i: (0, 0)),
            pl.BlockSpec((1, d), lambda i: (0, 0)),
            pl.BlockSpec((d, d), lambda i: (0, 0)),
            pl.BlockSpec((1, d), lambda i: (0, 0)),
        ],
        out_specs=[
            pl.BlockSpec((blk, d), lambda i: (i, 0)),
            pl.BlockSpec((blk, d), lambda i: (i, 0)),
        ],
        out_shape=[
            jax.ShapeDtypeStruct((n, d), jnp.float32),
            jax.ShapeDtypeStruct((n, d), jnp.float32),
        ],
    )(x, Wm, bm2, Wu2, bu2)


def _post_body(p0_ref, p1_ref, z_ref, wu1_ref, h_ref):
    agg = p0_ref[...] + p1_ref[...]
    h_ref[...] = jnp.maximum(
        jnp.dot(agg, wu1_ref[...], preferred_element_type=jnp.float32) + z_ref[...],
        0.0)


def _post(p0, p1, z, Wu1):
    n, d = z.shape
    blk = 2000
    grid = n // blk
    return pl.pallas_call(
        _post_body,
        grid=(grid,),
        in_specs=[
            pl.BlockSpec((blk, d), lambda i: (i, 0)),
            pl.BlockSpec((blk, d), lambda i: (i, 0)),
            pl.BlockSpec((blk, d), lambda i: (i, 0)),
            pl.BlockSpec((d, d), lambda i: (0, 0)),
        ],
        out_specs=pl.BlockSpec((blk, d), lambda i: (i, 0)),
        out_shape=jax.ShapeDtypeStruct((n, d), jnp.float32),
    )(p0, p1, z, Wu1)


def _sc_scatter(y, ei_r, zeros_pad, n, d, n_pad, k0, c0_chunks, base1, rem1,
                kmax):
    rows_out = n_pad // _NS

    def body(y_hbm, ei_hbm, zero_hbm, out_hbm, idx_s, idx_d, rows,
             agg_sh, sem):
        c = lax.axis_index("c")
        s = lax.axis_index("s")

        nch = jnp.where(c == 0, k0, base1 + jnp.where(s < rem1, 1, 0))
        start = pl.multiple_of(
            jnp.where(c == 0, s * k0,
                      c0_chunks + base1 * s + jnp.minimum(s, rem1)), 8)

        pltpu.sync_copy(zero_hbm.at[pl.ds(s * rows_out, rows_out)],
                        agg_sh.at[pl.ds(s * rows_out, rows_out)])
        pltpu.sync_copy(ei_hbm.at[0, pl.ds(start, kmax)], idx_s)
        pltpu.sync_copy(ei_hbm.at[1, pl.ds(start, kmax)], idx_d)
        plsc.subcore_barrier()

        def step(j, carry):
            pltpu.async_copy(y_hbm.at[idx_s.at[j]], rows, sem).wait()
            pltpu.sync_copy(rows, agg_sh.at[idx_d.at[j]], add=True)
            return carry

        lax.fori_loop(0, nch, step, 0, unroll=False)
        plsc.subcore_barrier()

        pltpu.sync_copy(agg_sh.at[pl.ds(s * rows_out, rows_out)],
                        out_hbm.at[c, pl.ds(s * rows_out, rows_out)])

    mesh = plsc.VectorSubcoreMesh(core_axis_name="c", subcore_axis_name="s")
    f = pl.kernel(
        body,
        out_type=jax.ShapeDtypeStruct((_NC, n_pad, d), jnp.float32),
        mesh=mesh,
        scratch_types=[
            pltpu.VMEM((kmax, _LANES), jnp.int32),
            pltpu.VMEM((kmax, _LANES), jnp.int32),
            pltpu.VMEM((_LANES, d), jnp.float32),
            pltpu.VMEM_SHARED((n_pad, d), jnp.float32),
            pltpu.SemaphoreType.DMA,
        ],
    )
    return f(y, ei_r, zeros_pad)


def kernel(x, edge_index, Wm, bm, Wu, bu):
    n, d = x.shape
    e = edge_index.shape[1]

    p = -(-e // (_LANES * 128)) * 128
    c0_chunks = max(_NS * 8, int(round(_F0 * p / 128)) * 128)
    c1_chunks = p - c0_chunks
    k0 = c0_chunks // _NS
    base1 = c1_chunks // _NS
    rem1 = 0
    kmax = max(k0, base1)
    p_pad = c0_chunks + base1 * (_NS - 1) + kmax
    n_pad = -(-(n + 1) // (_NS * 8)) * (_NS * 8)

    pad = p_pad * _LANES - e
    ar = jnp.arange(pad, dtype=jnp.int32)
    pad_cols = jnp.stack([ar % n, n + ar % (n_pad - n)])
    ei_r = jnp.concatenate([edge_index, pad_cols], axis=1).reshape(
        2, p_pad, _LANES)
    zeros_pad = jnp.zeros((n_pad, d), jnp.float32)

    bm2 = bm.reshape(1, d)
    bu2 = bu.reshape(1, d)
    Wu1 = Wu[:d]
    Wu2 = Wu[d:]

    y, z = _pre(x, Wm, bm2, Wu2, bu2)
    parts = _sc_scatter(y, ei_r, zeros_pad, n, d, n_pad, k0, c0_chunks,
                        base1, rem1, kmax)
    h = _post(parts[0, :n], parts[1, :n], z, Wu1)
    return h


# speedup vs baseline: 2.6181x; 1.1671x over previous
import functools

import jax
import jax.numpy as jnp
from jax import lax
from jax.experimental import pallas as pl
from jax.experimental.pallas import tpu as pltpu
from jax.experimental.pallas import tpu_sc as plsc

_NC = 2
_NS = 16
_NW = _NC * _NS
_LANES = 128
_F0 = 0.50


def _pre_body(x_ref, wm_ref, bm_ref, wu2_ref, bu_ref, y_ref, z_ref):
    xb = x_ref[...]
    y_ref[...] = jnp.maximum(
        jnp.dot(xb, wm_ref[...], preferred_element_type=jnp.float32) + bm_ref[...],
        0.0)
    z_ref[...] = jnp.dot(xb, wu2_ref[...], preferred_element_type=jnp.float32) + bu_ref[...]


def _pre(x, Wm, bm2, Wu2, bu2):
    n, d = x.shape
    blk = 2000
    grid = n // blk
    return pl.pallas_call(
        _pre_body,
        grid=(grid,),
        in_specs=[
            pl.BlockSpec((blk, d), lambda i: (i, 0)),
            pl.BlockSpec((d, d), lambda i: (0, 0)),
            pl.BlockSpec((1, d), lambda i: (0, 0)),
            pl.BlockSpec((d, d), lambda i: (0, 0)),
            pl.BlockSpec((1, d), lambda i: (0, 0)),
        ],
        out_specs=[
            pl.BlockSpec((blk, d), lambda i: (i, 0)),
            pl.BlockSpec((blk, d), lambda i: (i, 0)),
        ],
        out_shape=[
            jax.ShapeDtypeStruct((n, d), jnp.float32),
            jax.ShapeDtypeStruct((n, d), jnp.float32),
        ],
    )(x, Wm, bm2, Wu2, bu2)


def _post_body(p0_ref, p1_ref, z_ref, wu1_ref, h_ref):
    agg = p0_ref[0] + p1_ref[0]
    h_ref[...] = jnp.maximum(
        jnp.dot(agg, wu1_ref[...], preferred_element_type=jnp.float32) + z_ref[...],
        0.0)


def _post(parts, z, Wu1):
    n, d = z.shape
    blk = 2000
    grid = n // blk
    return pl.pallas_call(
        _post_body,
        grid=(grid,),
        in_specs=[
            pl.BlockSpec((1, blk, d), lambda i: (0, i, 0)),
            pl.BlockSpec((1, blk, d), lambda i: (1, i, 0)),
            pl.BlockSpec((blk, d), lambda i: (i, 0)),
            pl.BlockSpec((d, d), lambda i: (0, 0)),
        ],
        out_specs=pl.BlockSpec((blk, d), lambda i: (i, 0)),
        out_shape=jax.ShapeDtypeStruct((n, d), jnp.float32),
    )(parts, parts, z, Wu1)


def _sc_scatter(y, ei_r, zeros_pad, n, d, n_pad, k0, c0_chunks, base1, kb):
    rows_out = n_pad // _NS

    def body(y_hbm, ei_hbm, zero_hbm, out_hbm, idx_s, idx_d, rows,
             agg_sh, sem):
        c = lax.axis_index("c")
        s = lax.axis_index("s")

        nch = jnp.where(c == 0, k0, base1)
        start = pl.multiple_of(
            jnp.where(c == 0, s * k0, c0_chunks + base1 * s), 8)

        pltpu.sync_copy(zero_hbm.at[pl.ds(s * rows_out, rows_out)],
                        agg_sh.at[pl.ds(s * rows_out, rows_out)])
        pltpu.sync_copy(ei_hbm.at[0, pl.ds(start, kb)], idx_s)
        pltpu.sync_copy(ei_hbm.at[1, pl.ds(start, kb)], idx_d)
        plsc.subcore_barrier()

        def step(j, carry):
            pltpu.async_copy(y_hbm.at[idx_s.at[j]], rows, sem).wait()
            pltpu.sync_copy(rows, agg_sh.at[idx_d.at[j]], add=True)
            return carry

        lax.fori_loop(0, nch, step, 0, unroll=False)
        plsc.subcore_barrier()

        pltpu.sync_copy(agg_sh.at[pl.ds(s * rows_out, rows_out)],
                        out_hbm.at[c, pl.ds(s * rows_out, rows_out)])

    mesh = plsc.VectorSubcoreMesh(core_axis_name="c", subcore_axis_name="s")
    f = pl.kernel(
        body,
        out_type=jax.ShapeDtypeStruct((_NC, n_pad, d), jnp.float32),
        mesh=mesh,
        scratch_types=[
            pltpu.VMEM((kb, _LANES), jnp.int32),
            pltpu.VMEM((kb, _LANES), jnp.int32),
            pltpu.VMEM((_LANES, d), jnp.float32),
            pltpu.VMEM_SHARED((n_pad, d), jnp.float32),
            pltpu.SemaphoreType.DMA,
        ],
    )
    return f(y, ei_r, zeros_pad)


def kernel(x, edge_index, Wm, bm, Wu, bu):
    n, d = x.shape
    e = edge_index.shape[1]

    p = -(-e // (_LANES * 128)) * 128
    c0_chunks = min(p - _NS * 8,
                    max(_NS * 8, int(round(_F0 * p / 128)) * 128))
    c1_chunks = p - c0_chunks
    k0 = c0_chunks // _NS
    base1 = c1_chunks // _NS
    kb = max(k0, base1)
    n_pad = -(-(n + 1) // (_NS * 8)) * (_NS * 8)

    pad = p * _LANES - e
    ar = jnp.arange(pad, dtype=jnp.int32)
    pad_cols = jnp.stack([ar % n, n + ar % (n_pad - n)])
    ei_r = jnp.concatenate([edge_index, pad_cols], axis=1).reshape(
        2, p, _LANES)
    zeros_pad = jnp.zeros((n_pad, d), jnp.float32)

    bm2 = bm.reshape(1, d)
    bu2 = bu.reshape(1, d)
    Wu1 = Wu[:d]
    Wu2 = Wu[d:]

    y, z = _pre(x, Wm, bm2, Wu2, bu2)
    parts = _sc_scatter(y, ei_r, zeros_pad, n, d, n_pad, k0, c0_chunks,
                        base1, kb)
    h = _post(parts, z, Wu1)
    return h


# speedup vs baseline: 3.7209x; 1.4212x over previous
import functools

import jax
import jax.numpy as jnp
from jax import lax
from jax.experimental import pallas as pl
from jax.experimental.pallas import tpu as pltpu
from jax.experimental.pallas import tpu_sc as plsc

_NC = 2
_NS = 16
_NW = _NC * _NS
_LANES = 128
_F0 = 0.50
_NB = 2
_G = 40


def _pre_body(x_ref, wm_ref, bm_ref, wu2_ref, bu_ref, y_ref, z_ref):
    xb = x_ref[...]
    y_ref[...] = jnp.maximum(
        jnp.dot(xb, wm_ref[...], preferred_element_type=jnp.float32) + bm_ref[...],
        0.0)
    z_ref[...] = jnp.dot(xb, wu2_ref[...], preferred_element_type=jnp.float32) + bu_ref[...]


def _pre(x, Wm, bm2, Wu2, bu2):
    n, d = x.shape
    blk = 2000
    grid = n // blk
    return pl.pallas_call(
        _pre_body,
        grid=(grid,),
        in_specs=[
            pl.BlockSpec((blk, d), lambda i: (i, 0)),
            pl.BlockSpec((d, d), lambda i: (0, 0)),
            pl.BlockSpec((1, d), lambda i: (0, 0)),
            pl.BlockSpec((d, d), lambda i: (0, 0)),
            pl.BlockSpec((1, d), lambda i: (0, 0)),
        ],
        out_specs=[
            pl.BlockSpec((blk, d), lambda i: (i, 0)),
            pl.BlockSpec((blk, d), lambda i: (i, 0)),
        ],
        out_shape=[
            jax.ShapeDtypeStruct((n, d), jnp.float32),
            jax.ShapeDtypeStruct((n, d), jnp.float32),
        ],
    )(x, Wm, bm2, Wu2, bu2)


def _post_body(p0_ref, p1_ref, z_ref, wu1_ref, h_ref):
    agg = p0_ref[0] + p1_ref[0]
    h_ref[...] = jnp.maximum(
        jnp.dot(agg, wu1_ref[...], preferred_element_type=jnp.float32) + z_ref[...],
        0.0)


def _post(parts, z, Wu1):
    n, d = z.shape
    blk = 2000
    grid = n // blk
    return pl.pallas_call(
        _post_body,
        grid=(grid,),
        in_specs=[
            pl.BlockSpec((1, blk, d), lambda i: (0, i, 0)),
            pl.BlockSpec((1, blk, d), lambda i: (1, i, 0)),
            pl.BlockSpec((blk, d), lambda i: (i, 0)),
            pl.BlockSpec((d, d), lambda i: (0, 0)),
        ],
        out_specs=pl.BlockSpec((blk, d), lambda i: (i, 0)),
        out_shape=jax.ShapeDtypeStruct((n, d), jnp.float32),
    )(parts, parts, z, Wu1)


def _sc_scatter(y, ei_r, zeros_pad, n, d, n_pad, k0, c0_chunks, base1, kb):
    rows_out = n_pad // _NS

    def body(y_hbm, ei_hbm, zero_hbm, out_hbm, idx_s, idx_d, rows,
             agg_sh, *sems):
        c = lax.axis_index("c")
        s = lax.axis_index("s")

        nch = jnp.where(c == 0, k0, base1)
        start = pl.multiple_of(
            jnp.where(c == 0, s * k0, c0_chunks + base1 * s), 8)

        pltpu.sync_copy(zero_hbm.at[pl.ds(s * rows_out, rows_out)],
                        agg_sh.at[pl.ds(s * rows_out, rows_out)])
        plsc.subcore_barrier()

        def group(g, carry):
            gb = start + g * _G
            pltpu.sync_copy(ei_hbm.at[0, pl.ds(gb, _G)], idx_s)
            pltpu.sync_copy(ei_hbm.at[1, pl.ds(gb, _G)], idx_d)
            for b in range(_NB):
                pltpu.async_copy(y_hbm.at[idx_s.at[b]], rows.at[b], sems[b])

            def step(i, c2):
                for b in range(_NB):
                    jj = i * _NB + b
                    pltpu.make_async_copy(y_hbm.at[idx_s.at[jj]],
                                          rows.at[b], sems[b]).wait()
                    pltpu.sync_copy(rows.at[b], agg_sh.at[idx_d.at[jj]],
                                    add=True)
                    nj = jj + _NB

                    @pl.when(nj < _G)
                    def _():
                        pltpu.async_copy(y_hbm.at[idx_s.at[nj]],
                                         rows.at[b], sems[b])
                return c2

            lax.fori_loop(0, _G // _NB, step, 0, unroll=False)
            return carry

        lax.fori_loop(0, nch // _G, group, 0, unroll=False)
        plsc.subcore_barrier()

        pltpu.sync_copy(agg_sh.at[pl.ds(s * rows_out, rows_out)],
                        out_hbm.at[c, pl.ds(s * rows_out, rows_out)])

    mesh = plsc.VectorSubcoreMesh(core_axis_name="c", subcore_axis_name="s")
    f = pl.kernel(
        body,
        out_type=jax.ShapeDtypeStruct((_NC, n_pad, d), jnp.float32),
        mesh=mesh,
        scratch_types=[
            pltpu.VMEM((_G, _LANES), jnp.int32),
            pltpu.VMEM((_G, _LANES), jnp.int32),
            pltpu.VMEM((_NB, _LANES, d), jnp.float32),
            pltpu.VMEM_SHARED((n_pad, d), jnp.float32),
        ] + [pltpu.SemaphoreType.DMA] * _NB,
    )
    return f(y, ei_r, zeros_pad)


def kernel(x, edge_index, Wm, bm, Wu, bu):
    n, d = x.shape
    e = edge_index.shape[1]

    p = -(-e // (_LANES * 128)) * 128
    c0_chunks = min(p - _NS * 8,
                    max(_NS * 8, int(round(_F0 * p / 128)) * 128))
    c1_chunks = p - c0_chunks
    k0 = c0_chunks // _NS
    base1 = c1_chunks // _NS
    kb = max(k0, base1)
    n_pad = -(-(n + 1) // (_NS * 8)) * (_NS * 8)

    pad = p * _LANES - e
    ar = jnp.arange(pad, dtype=jnp.int32)
    pad_cols = jnp.stack([ar % n, n + ar % (n_pad - n)])
    ei_r = jnp.concatenate([edge_index, pad_cols], axis=1).reshape(
        2, p, _LANES)
    zeros_pad = jnp.zeros((n_pad, d), jnp.float32)

    bm2 = bm.reshape(1, d)
    bu2 = bu.reshape(1, d)
    Wu1 = Wu[:d]
    Wu2 = Wu[d:]

    y, z = _pre(x, Wm, bm2, Wu2, bu2)
    parts = _sc_scatter(y, ei_r, zeros_pad, n, d, n_pad, k0, c0_chunks,
                        base1, kb)
    h = _post(parts, z, Wu1)
    return h


# speedup vs baseline: 3.7554x; 1.0093x over previous
import functools

import jax
import jax.numpy as jnp
from jax import lax
from jax.experimental import pallas as pl
from jax.experimental.pallas import tpu as pltpu
from jax.experimental.pallas import tpu_sc as plsc

_NC = 2
_NS = 16
_NW = _NC * _NS
_LANES = 128
_F0 = 0.50
_NB = 2
_G = 40


def _mm_relu_body(x_ref, w_ref, b_ref, o_ref):
    o_ref[...] = jnp.maximum(
        jnp.dot(x_ref[...], w_ref[...], preferred_element_type=jnp.float32)
        + b_ref[...], 0.0)


def _mm_body(x_ref, w_ref, b_ref, o_ref):
    o_ref[...] = (
        jnp.dot(x_ref[...], w_ref[...], preferred_element_type=jnp.float32)
        + b_ref[...])


def _mm(body, x, W, b2):
    n, d = x.shape
    blk = 2000
    grid = n // blk
    return pl.pallas_call(
        body,
        grid=(grid,),
        in_specs=[
            pl.BlockSpec((blk, d), lambda i: (i, 0)),
            pl.BlockSpec((d, d), lambda i: (0, 0)),
            pl.BlockSpec((1, d), lambda i: (0, 0)),
        ],
        out_specs=pl.BlockSpec((blk, d), lambda i: (i, 0)),
        out_shape=jax.ShapeDtypeStruct((n, d), jnp.float32),
    )(x, W, b2)


def _post_body(p0_ref, p1_ref, z_ref, wu1_ref, h_ref):
    agg = p0_ref[0] + p1_ref[0]
    h_ref[...] = jnp.maximum(
        jnp.dot(agg, wu1_ref[...], preferred_element_type=jnp.float32) + z_ref[...],
        0.0)


def _post(parts, z, Wu1):
    n, d = z.shape
    blk = 2000
    grid = n // blk
    return pl.pallas_call(
        _post_body,
        grid=(grid,),
        in_specs=[
            pl.BlockSpec((1, blk, d), lambda i: (0, i, 0)),
            pl.BlockSpec((1, blk, d), lambda i: (1, i, 0)),
            pl.BlockSpec((blk, d), lambda i: (i, 0)),
            pl.BlockSpec((d, d), lambda i: (0, 0)),
        ],
        out_specs=pl.BlockSpec((blk, d), lambda i: (i, 0)),
        out_shape=jax.ShapeDtypeStruct((n, d), jnp.float32),
    )(parts, parts, z, Wu1)


def _sc_scatter(y, ei_r, zeros_pad, n, d, n_pad, k0, c0_chunks, base1, kb):
    rows_out = n_pad // _NS

    def body(y_hbm, ei_hbm, zero_hbm, out_hbm, idx_s, idx_d, rows,
             agg_sh, *sems):
        c = lax.axis_index("c")
        s = lax.axis_index("s")

        nch = jnp.where(c == 0, k0, base1)
        start = pl.multiple_of(
            jnp.where(c == 0, s * k0, c0_chunks + base1 * s), 8)

        pltpu.sync_copy(ei_hbm.at[0, pl.ds(start, _G)], idx_s)
        pltpu.sync_copy(ei_hbm.at[1, pl.ds(start, _G)], idx_d)
        for b in range(_NB):
            pltpu.async_copy(y_hbm.at[idx_s.at[b]], rows.at[b], sems[b])
        pltpu.sync_copy(zero_hbm,
                        agg_sh.at[pl.ds(s * rows_out, rows_out)])
        plsc.subcore_barrier()

        def group(g, carry):
            @pl.when(g != 0)
            def _():
                gb = start + g * _G
                pltpu.sync_copy(ei_hbm.at[0, pl.ds(gb, _G)], idx_s)
                pltpu.sync_copy(ei_hbm.at[1, pl.ds(gb, _G)], idx_d)
                for b in range(_NB):
                    pltpu.async_copy(y_hbm.at[idx_s.at[b]], rows.at[b],
                                     sems[b])

            def step(i, c2):
                for b in range(_NB):
                    jj = i * _NB + b
                    pltpu.make_async_copy(y_hbm.at[idx_s.at[jj]],
                                          rows.at[b], sems[b]).wait()
                    pltpu.sync_copy(rows.at[b], agg_sh.at[idx_d.at[jj]],
                                    add=True)
                    nj = jj + _NB

                    @pl.when(nj < _G)
                    def _():
                        pltpu.async_copy(y_hbm.at[idx_s.at[nj]],
                                         rows.at[b], sems[b])
                return c2

            lax.fori_loop(0, _G // _NB, step, 0, unroll=False)
            return carry

        lax.fori_loop(0, nch // _G, group, 0, unroll=False)
        plsc.subcore_barrier()

        pltpu.sync_copy(agg_sh.at[pl.ds(s * rows_out, rows_out)],
                        out_hbm.at[c, pl.ds(s * rows_out, rows_out)])

    mesh = plsc.VectorSubcoreMesh(core_axis_name="c", subcore_axis_name="s")
    f = pl.kernel(
        body,
        out_type=jax.ShapeDtypeStruct((_NC, n_pad, d), jnp.float32),
        mesh=mesh,
        scratch_types=[
            pltpu.VMEM((_G, _LANES), jnp.int32),
            pltpu.VMEM((_G, _LANES), jnp.int32),
            pltpu.VMEM((_NB, _LANES, d), jnp.float32),
            pltpu.VMEM_SHARED((n_pad, d), jnp.float32),
        ] + [pltpu.SemaphoreType.DMA] * _NB,
    )
    return f(y, ei_r, zeros_pad)


def kernel(x, edge_index, Wm, bm, Wu, bu):
    n, d = x.shape
    e = edge_index.shape[1]

    p = -(-e // (_LANES * 128)) * 128
    c0_chunks = min(p - _NS * 8,
                    max(_NS * 8, int(round(_F0 * p / 128)) * 128))
    c1_chunks = p - c0_chunks
    k0 = c0_chunks // _NS
    base1 = c1_chunks // _NS
    kb = max(k0, base1)
    n_pad = -(-(n + 1) // (_NS * 8)) * (_NS * 8)

    pad = p * _LANES - e
    ar = jnp.arange(pad, dtype=jnp.int32)
    pad_cols = jnp.stack([ar % n, n + ar % (n_pad - n)])
    ei_r = jnp.concatenate([edge_index, pad_cols], axis=1).reshape(
        2, p, _LANES)
    zeros_pad = jnp.zeros((n_pad // _NS, d), jnp.float32)

    bm2 = bm.reshape(1, d)
    bu2 = bu.reshape(1, d)
    Wu1 = Wu[:d]
    Wu2 = Wu[d:]

    y = _mm(_mm_relu_body, x, Wm, bm2)
    z = _mm(_mm_body, x, Wu2, bu2)
    parts = _sc_scatter(y, ei_r, zeros_pad, n, d, n_pad, k0, c0_chunks,
                        base1, kb)
    h = _post(parts, z, Wu1)
    return h


# speedup vs baseline: 3.8556x; 1.0267x over previous
import functools

import jax
import jax.numpy as jnp
from jax import lax
from jax.experimental import pallas as pl
from jax.experimental.pallas import tpu as pltpu
from jax.experimental.pallas import tpu_sc as plsc

_NC = 2
_NS = 16
_NW = _NC * _NS
_LANES = 128
_F0 = 0.50
_NB = 2
_G = 40


def _pre_make(n, d, e, p, n_pad):
    er = e // _LANES
    pad_rows = p - er

    def body(x_ref, w_ref, b_ref, ei_ref, y_ref, es_ref, ed_ref, z0_ref):
        y_ref[...] = jnp.maximum(
            jnp.dot(x_ref[...], w_ref[...], preferred_element_type=jnp.float32)
            + b_ref[...], 0.0)

        @pl.when(pl.program_id(0) == 0)
        def _():
            z0_ref[...] = jnp.zeros_like(z0_ref)
            iw = lax.broadcasted_iota(jnp.int32, (pad_rows, _LANES), 0)
            il = lax.broadcasted_iota(jnp.int32, (pad_rows, _LANES), 1)
            flat = iw * _LANES + il
            es_ref[:er] = ei_ref[:er]
            es_ref[er:] = flat % n
            ed_ref[:er] = ei_ref[er:]
            ed_ref[er:] = n + flat % (n_pad - n)

    blk = 2000
    return pl.pallas_call(
        body,
        grid=(n // blk,),
        in_specs=[
            pl.BlockSpec((blk, d), lambda i: (i, 0)),
            pl.BlockSpec((d, d), lambda i: (0, 0)),
            pl.BlockSpec((1, d), lambda i: (0, 0)),
            pl.BlockSpec((2 * er, _LANES), lambda i: (0, 0)),
        ],
        out_specs=[
            pl.BlockSpec((blk, d), lambda i: (i, 0)),
            pl.BlockSpec((p, _LANES), lambda i: (0, 0)),
            pl.BlockSpec((p, _LANES), lambda i: (0, 0)),
            pl.BlockSpec((n_pad // _NS, d), lambda i: (0, 0)),
        ],
        out_shape=[
            jax.ShapeDtypeStruct((n, d), jnp.float32),
            jax.ShapeDtypeStruct((p, _LANES), jnp.int32),
            jax.ShapeDtypeStruct((p, _LANES), jnp.int32),
            jax.ShapeDtypeStruct((n_pad // _NS, d), jnp.float32),
        ],
    )


def _mm_body(x_ref, w_ref, b_ref, o_ref):
    o_ref[...] = (
        jnp.dot(x_ref[...], w_ref[...], preferred_element_type=jnp.float32)
        + b_ref[...])


def _mm(body, x, W, b2):
    n, d = x.shape
    blk = 2000
    grid = n // blk
    return pl.pallas_call(
        body,
        grid=(grid,),
        in_specs=[
            pl.BlockSpec((blk, d), lambda i: (i, 0)),
            pl.BlockSpec((d, d), lambda i: (0, 0)),
            pl.BlockSpec((1, d), lambda i: (0, 0)),
        ],
        out_specs=pl.BlockSpec((blk, d), lambda i: (i, 0)),
        out_shape=jax.ShapeDtypeStruct((n, d), jnp.float32),
    )(x, W, b2)


def _post_body(p0_ref, p1_ref, z_ref, wu1_ref, h_ref):
    agg = p0_ref[0] + p1_ref[0]
    h_ref[...] = jnp.maximum(
        jnp.dot(agg, wu1_ref[...], preferred_element_type=jnp.float32) + z_ref[...],
        0.0)


def _post(parts, z, Wu1):
    n, d = z.shape
    blk = 2000
    grid = n // blk
    return pl.pallas_call(
        _post_body,
        grid=(grid,),
        in_specs=[
            pl.BlockSpec((1, blk, d), lambda i: (0, i, 0)),
            pl.BlockSpec((1, blk, d), lambda i: (1, i, 0)),
            pl.BlockSpec((blk, d), lambda i: (i, 0)),
            pl.BlockSpec((d, d), lambda i: (0, 0)),
        ],
        out_specs=pl.BlockSpec((blk, d), lambda i: (i, 0)),
        out_shape=jax.ShapeDtypeStruct((n, d), jnp.float32),
    )(parts, parts, z, Wu1)


def _sc_scatter(y, es, ed, zeros_pad, n, d, n_pad, k0, c0_chunks, base1, kb):
    rows_out = n_pad // _NS

    def body(y_hbm, es_hbm, ed_hbm, zero_hbm, out_hbm, idx_s, idx_d, rows,
             agg_sh, *sems):
        c = lax.axis_index("c")
        s = lax.axis_index("s")

        nch = jnp.where(c == 0, k0, base1)
        start = pl.multiple_of(
            jnp.where(c == 0, s * k0, c0_chunks + base1 * s), 8)

        pltpu.sync_copy(es_hbm.at[pl.ds(start, _G)], idx_s)
        pltpu.sync_copy(ed_hbm.at[pl.ds(start, _G)], idx_d)
        for b in range(_NB):
            pltpu.async_copy(y_hbm.at[idx_s.at[b]], rows.at[b], sems[b])
        pltpu.sync_copy(zero_hbm,
                        agg_sh.at[pl.ds(s * rows_out, rows_out)])
        plsc.subcore_barrier()

        def group(g, carry):
            @pl.when(g != 0)
            def _():
                gb = start + g * _G
                pltpu.sync_copy(es_hbm.at[pl.ds(gb, _G)], idx_s)
                pltpu.sync_copy(ed_hbm.at[pl.ds(gb, _G)], idx_d)
                for b in range(_NB):
                    pltpu.async_copy(y_hbm.at[idx_s.at[b]], rows.at[b],
                                     sems[b])

            def step(i, c2):
                for b in range(_NB):
                    jj = i * _NB + b
                    pltpu.make_async_copy(y_hbm.at[idx_s.at[jj]],
                                          rows.at[b], sems[b]).wait()
                    pltpu.sync_copy(rows.at[b], agg_sh.at[idx_d.at[jj]],
                                    add=True)
                    nj = jj + _NB

                    @pl.when(nj < _G)
                    def _():
                        pltpu.async_copy(y_hbm.at[idx_s.at[nj]],
                                         rows.at[b], sems[b])
                return c2

            lax.fori_loop(0, _G // _NB, step, 0, unroll=False)
            return carry

        lax.fori_loop(0, nch // _G, group, 0, unroll=False)
        plsc.subcore_barrier()

        pltpu.sync_copy(agg_sh.at[pl.ds(s * rows_out, rows_out)],
                        out_hbm.at[c, pl.ds(s * rows_out, rows_out)])

    mesh = plsc.VectorSubcoreMesh(core_axis_name="c", subcore_axis_name="s")
    f = pl.kernel(
        body,
        out_type=jax.ShapeDtypeStruct((_NC, n_pad, d), jnp.float32),
        mesh=mesh,
        scratch_types=[
            pltpu.VMEM((_G, _LANES), jnp.int32),
            pltpu.VMEM((_G, _LANES), jnp.int32),
            pltpu.VMEM((_NB, _LANES, d), jnp.float32),
            pltpu.VMEM_SHARED((n_pad, d), jnp.float32),
        ] + [pltpu.SemaphoreType.DMA] * _NB,
    )
    return f(y, es, ed, zeros_pad)


def kernel(x, edge_index, Wm, bm, Wu, bu):
    n, d = x.shape
    e = edge_index.shape[1]

    p = -(-e // (_LANES * 128)) * 128
    c0_chunks = min(p - _NS * 8,
                    max(_NS * 8, int(round(_F0 * p / 128)) * 128))
    c1_chunks = p - c0_chunks
    k0 = c0_chunks // _NS
    base1 = c1_chunks // _NS
    kb = max(k0, base1)
    n_pad = -(-(n + 1) // (_NS * 8)) * (_NS * 8)

    bm2 = bm.reshape(1, d)
    bu2 = bu.reshape(1, d)
    Wu1 = Wu[:d]
    Wu2 = Wu[d:]

    ei_rows = edge_index.reshape(2 * (e // _LANES), _LANES)
    y, es, ed, zeros_pad = _pre_make(n, d, e, p, n_pad)(x, Wm, bm2, ei_rows)
    z = _mm(_mm_body, x, Wu2, bu2)
    parts = _sc_scatter(y, es, ed, zeros_pad, n, d, n_pad, k0, c0_chunks,
                        base1, kb)
    h = _post(parts, z, Wu1)
    return h


# speedup vs baseline: 3.8620x; 1.0017x over previous
import jax
import jax.numpy as jnp
from jax import lax
from jax.experimental import pallas as pl
from jax.experimental.pallas import tpu as pltpu
from jax.experimental.pallas import tpu_sc as plsc

_NC = 2
_NS = 16
_NW = _NC * _NS
_LANES = 128
_F0 = 0.50
_NB = 2
_G = 40


def _pre_make(n, d, e, p, n_pad):
    er = e // _LANES
    pad_rows = p - er

    def body(x_ref, w_ref, b_ref, ei_ref, y_ref, es_ref, ed_ref, z0_ref):
        y_ref[...] = jnp.maximum(
            jnp.dot(x_ref[...], w_ref[...], preferred_element_type=jnp.float32)
            + b_ref[...], 0.0)

        @pl.when(pl.program_id(0) == 0)
        def _():
            z0_ref[...] = jnp.zeros_like(z0_ref)
            iw = lax.broadcasted_iota(jnp.int32, (pad_rows, _LANES), 0)
            il = lax.broadcasted_iota(jnp.int32, (pad_rows, _LANES), 1)
            flat = iw * _LANES + il
            es_ref[:er] = ei_ref[:er]
            es_ref[er:] = flat % n
            ed_ref[:er] = ei_ref[er:]
            ed_ref[er:] = n + flat % (n_pad - n)

    blk = 2000
    return pl.pallas_call(
        body,
        grid=(n // blk,),
        in_specs=[
            pl.BlockSpec((blk, d), lambda i: (i, 0)),
            pl.BlockSpec((d, d), lambda i: (0, 0)),
            pl.BlockSpec((1, d), lambda i: (0, 0)),
            pl.BlockSpec((2 * er, _LANES), lambda i: (0, 0)),
        ],
        out_specs=[
            pl.BlockSpec((blk, d), lambda i: (i, 0)),
            pl.BlockSpec((p, _LANES), lambda i: (0, 0)),
            pl.BlockSpec((p, _LANES), lambda i: (0, 0)),
            pl.BlockSpec((n_pad // _NS, d), lambda i: (0, 0)),
        ],
        out_shape=[
            jax.ShapeDtypeStruct((n, d), jnp.float32),
            jax.ShapeDtypeStruct((p, _LANES), jnp.int32),
            jax.ShapeDtypeStruct((p, _LANES), jnp.int32),
            jax.ShapeDtypeStruct((n_pad // _NS, d), jnp.float32),
        ],
    )


def _mm_body(x_ref, w_ref, b_ref, o_ref):
    o_ref[...] = (
        jnp.dot(x_ref[...], w_ref[...], preferred_element_type=jnp.float32)
        + b_ref[...])


def _mm(body, x, W, b2):
    n, d = x.shape
    blk = 2000
    grid = n // blk
    return pl.pallas_call(
        body,
        grid=(grid,),
        in_specs=[
            pl.BlockSpec((blk, d), lambda i: (i, 0)),
            pl.BlockSpec((d, d), lambda i: (0, 0)),
            pl.BlockSpec((1, d), lambda i: (0, 0)),
        ],
        out_specs=pl.BlockSpec((blk, d), lambda i: (i, 0)),
        out_shape=jax.ShapeDtypeStruct((n, d), jnp.float32),
    )(x, W, b2)


def _post_body(p0_ref, p1_ref, z_ref, wu1_ref, h_ref):
    agg = p0_ref[0] + p1_ref[0]
    h_ref[...] = jnp.maximum(
        jnp.dot(agg, wu1_ref[...], preferred_element_type=jnp.float32) + z_ref[...],
        0.0)


def _post(parts, z, Wu1):
    n, d = z.shape
    blk = 2000
    grid = n // blk
    return pl.pallas_call(
        _post_body,
        grid=(grid,),
        in_specs=[
            pl.BlockSpec((1, blk, d), lambda i: (0, i, 0)),
            pl.BlockSpec((1, blk, d), lambda i: (1, i, 0)),
            pl.BlockSpec((blk, d), lambda i: (i, 0)),
            pl.BlockSpec((d, d), lambda i: (0, 0)),
        ],
        out_specs=pl.BlockSpec((blk, d), lambda i: (i, 0)),
        out_shape=jax.ShapeDtypeStruct((n, d), jnp.float32),
    )(parts, parts, z, Wu1)


def _sc_scatter(y, es, ed, zeros_pad, n, d, n_pad, k0, c0_chunks, base1):
    rows_out = n_pad // _NS

    def body(y_hbm, es_hbm, ed_hbm, zero_hbm, out_hbm, idx_s, idx_d, rows,
             agg_sh, *sems):
        c = lax.axis_index("c")
        s = lax.axis_index("s")

        nch = jnp.where(c == 0, k0, base1)
        start = pl.multiple_of(
            jnp.where(c == 0, s * k0, c0_chunks + base1 * s), 8)

        pltpu.sync_copy(es_hbm.at[pl.ds(start, _G)], idx_s)
        pltpu.sync_copy(ed_hbm.at[pl.ds(start, _G)], idx_d)
        for b in range(_NB):
            pltpu.async_copy(y_hbm.at[idx_s.at[b]], rows.at[b], sems[b])
        pltpu.sync_copy(zero_hbm,
                        agg_sh.at[pl.ds(s * rows_out, rows_out)])
        plsc.subcore_barrier()

        def group(g, carry):
            @pl.when(g != 0)
            def _():
                gb = start + g * _G
                pltpu.sync_copy(es_hbm.at[pl.ds(gb, _G)], idx_s)
                pltpu.sync_copy(ed_hbm.at[pl.ds(gb, _G)], idx_d)
                for b in range(_NB):
                    pltpu.async_copy(y_hbm.at[idx_s.at[b]], rows.at[b],
                                     sems[b])

            def step(i, c2):
                for b in range(_NB):
                    jj = i * _NB + b
                    pltpu.make_async_copy(y_hbm.at[idx_s.at[jj]],
                                          rows.at[b], sems[b]).wait()
                    pltpu.sync_copy(rows.at[b], agg_sh.at[idx_d.at[jj]],
                                    add=True)
                    nj = jj + _NB

                    @pl.when(nj < _G)
                    def _():
                        pltpu.async_copy(y_hbm.at[idx_s.at[nj]],
                                         rows.at[b], sems[b])
                return c2

            lax.fori_loop(0, _G // _NB, step, 0, unroll=False)
            return carry

        lax.fori_loop(0, nch // _G, group, 0, unroll=False)
        plsc.subcore_barrier()

        pltpu.sync_copy(agg_sh.at[pl.ds(s * rows_out, rows_out)],
                        out_hbm.at[c, pl.ds(s * rows_out, rows_out)])

    mesh = plsc.VectorSubcoreMesh(core_axis_name="c", subcore_axis_name="s")
    f = pl.kernel(
        body,
        out_type=jax.ShapeDtypeStruct((_NC, n_pad, d), jnp.float32),
        mesh=mesh,
        scratch_types=[
            pltpu.VMEM((_G, _LANES), jnp.int32),
            pltpu.VMEM((_G, _LANES), jnp.int32),
            pltpu.VMEM((_NB, _LANES, d), jnp.float32),
            pltpu.VMEM_SHARED((n_pad, d), jnp.float32),
        ] + [pltpu.SemaphoreType.DMA] * _NB,
    )
    return f(y, es, ed, zeros_pad)


def kernel(x, edge_index, Wm, bm, Wu, bu):
    n, d = x.shape
    e = edge_index.shape[1]

    p = -(-e // (_LANES * 128)) * 128
    c0_chunks = min(p - _NS * 8,
                    max(_NS * 8, int(round(_F0 * p / 128)) * 128))
    c1_chunks = p - c0_chunks
    k0 = c0_chunks // _NS
    base1 = c1_chunks // _NS
    n_pad = -(-(n + 1) // (_NS * 8)) * (_NS * 8)

    bm2 = bm.reshape(1, d)
    bu2 = bu.reshape(1, d)
    Wu1 = Wu[:d]
    Wu2 = Wu[d:]

    ei_rows = edge_index.reshape(2 * (e // _LANES), _LANES)
    y, es, ed, zeros_pad = _pre_make(n, d, e, p, n_pad)(x, Wm, bm2, ei_rows)
    z = _mm(_mm_body, x, Wu2, bu2)
    parts = _sc_scatter(y, es, ed, zeros_pad, n, d, n_pad, k0, c0_chunks,
                        base1)
    h = _post(parts, z, Wu1)
    return h
